# baseline (speedup 1.0000x reference)
import functools

import jax
import jax.numpy as jnp
from jax import lax
from jax.experimental import pallas as pl
from jax.experimental.pallas import tpu as pltpu
from jax.experimental.pallas import tpu_sc as plsc


def _mean_body(x_ref, out_ref, xbf_ref, *, T):
    x = x_ref[0]
    out_ref[0] = jnp.sum(x, axis=1) * (1.0 / T)
    xbf_ref[0] = x.astype(jnp.bfloat16)


def _select_body(s_ref, a_ref, vals_ref, idx_ref, apool_ref, *, K):
    s = s_ref[...]
    B, V = s.shape

    su = s[:, :, None]
    sv = s[:, None, :]
    u_iota = lax.broadcasted_iota(jnp.int32, (B, V, V), 1)
    v_iota = lax.broadcasted_iota(jnp.int32, (B, V, V), 2)
    beats = (su > sv) | ((su == sv) & (u_iota < v_iota))
    rank = jnp.sum(beats.astype(jnp.float32), axis=1)
    keep = rank < K

    rank_i = rank.astype(jnp.int32)
    r_iota = lax.broadcasted_iota(jnp.int32, (B, V, K), 2)
    hit = rank_i[:, :, None] == r_iota
    vals_ref[...] = jnp.sum(jnp.where(hit, s[:, :, None], 0.0), axis=1)

    tri = (lax.broadcasted_iota(jnp.int32, (V, V), 0)
           <= lax.broadcasted_iota(jnp.int32, (V, V), 1)).astype(jnp.float32)
    keep_f = keep.astype(jnp.float32)
    slot = jax.lax.dot_general(
        keep_f, tri, (((1,), (0,)), ((), ())),
        preferred_element_type=jnp.float32).astype(jnp.int32) - 1
    k_iota = lax.broadcasted_iota(jnp.int32, (B, V, K), 2)
    sel = (rank_i[:, :, None] < K) & (slot[:, :, None] == k_iota)
    v_in_iota = lax.broadcasted_iota(jnp.int32, (B, V, K), 1)
    idx_ref[...] = jnp.sum(jnp.where(sel, v_in_iota, 0), axis=1)

    selT = (rank_i[:, None, :] < K) & (slot[:, None, :] == lax.broadcasted_iota(
        jnp.int32, (B, K, V), 1))
    sel_f = sel.astype(jnp.float32)
    selT_f = selT.astype(jnp.float32)
    for bi in range(B):
        a_b = a_ref[bi]
        rows = jax.lax.dot_general(
            selT_f[bi], a_b, (((1,), (0,)), ((), ())),
            preferred_element_type=jnp.float32,
            precision=jax.lax.Precision.HIGHEST)
        apool_ref[bi] = jax.lax.dot_general(
            rows, sel_f[bi], (((1,), (0,)), ((), ())),
            preferred_element_type=jnp.float32,
            precision=jax.lax.Precision.HIGHEST)


def _make_sc_apool(B, V, K):
    NW = 32
    WPB = NW // B
    RPW = K // WPB
    NJ = K // 16
    mesh = plsc.VectorSubcoreMesh(core_axis_name="c", subcore_axis_name="s")

    import functools as _ft

    @_ft.partial(
        pl.kernel, mesh=mesh,
        compiler_params=pltpu.CompilerParams(needs_layout_passes=False),
        out_type=jax.ShapeDtypeStruct((B * K * K,), jnp.float32),
        scratch_types=[
            pltpu.VMEM((K,), jnp.int32),
            pltpu.VMEM((V * V,), jnp.float32),
            pltpu.VMEM((RPW * K,), jnp.float32),
        ],
    )
    def sc_apool(a_hbm, idx_hbm, out_hbm, idx_v, a_v, out_v):
        wid = lax.axis_index("s") * 2 + lax.axis_index("c")
        bi = wid // WPB
        k0 = (wid % WPB) * RPW

        pltpu.sync_copy(idx_hbm.at[bi], idx_v)
        pltpu.sync_copy(a_hbm.at[bi], a_v)
        cols = [idx_v[pl.ds(16 * j, 16)] for j in range(NJ)]

        for m in range(RPW):
            sp = jnp.zeros((16,), jnp.int32) + (k0 + m)
            rowv = plsc.load_gather(idx_v, [sp]) * V
            for j in range(NJ):
                g = plsc.load_gather(a_v, [rowv + cols[j]])
                out_v[pl.ds(m * K + 16 * j, 16)] = g
        pltpu.sync_copy(out_v, out_hbm.at[pl.ds((bi * K + k0) * K, RPW * K)])

    return sc_apool


def _make_sc_feature(B, CT, V, K):
    NW = 32
    WPB = NW // B
    ROWS_W = CT // WPB
    R = 128
    NT = ROWS_W // R
    NJ = K // 16
    mesh = plsc.VectorSubcoreMesh(core_axis_name="c", subcore_axis_name="s")

    import functools as _ft

    @_ft.partial(
        pl.kernel, mesh=mesh,
        compiler_params=pltpu.CompilerParams(needs_layout_passes=False),
        out_type=jax.ShapeDtypeStruct((B * CT * K,), jnp.float32),
        scratch_types=[
            pltpu.VMEM((K,), jnp.int32),
            pltpu.VMEM((K,), jnp.float32),
            pltpu.VMEM((R * V,), jnp.float32),
            pltpu.VMEM((R * V,), jnp.float32),
            pltpu.VMEM((R * K,), jnp.float32),
            pltpu.VMEM((R * K,), jnp.float32),
            pltpu.SemaphoreType.DMA,
            pltpu.SemaphoreType.DMA,
            pltpu.SemaphoreType.DMA,
            pltpu.SemaphoreType.DMA,
        ],
    )
    def sc_feature(x_hbm, idx_hbm, vals_hbm, out_hbm,
                   idx_v, vals_v, in0, in1, ou0, ou1, si0, si1, so0, so1):
        wid = lax.axis_index("s") * 2 + lax.axis_index("c")
        bi = wid // WPB
        row0 = (wid % WPB) * ROWS_W

        pltpu.sync_copy(idx_hbm.at[bi], idx_v)
        pltpu.sync_copy(vals_hbm.at[bi], vals_v)
        cols = [idx_v[pl.ds(16 * j, 16)] for j in range(NJ)]
        valv = [vals_v[pl.ds(16 * j, 16)] for j in range(NJ)]

        inbufs, in_sems = [in0, in1], [si0, si1]
        oubufs, out_sems = [ou0, ou1], [so0, so1]
        in_cp = {}
        out_cp = {}

        def start_in(t):
            in_cp[t] = pltpu.async_copy(
                x_hbm.at[bi, pl.ds((row0 + t * R) * V, R * V)],
                inbufs[t % 2], in_sems[t % 2])

        def start_out(t):
            out_cp[t] = pltpu.async_copy(
                oubufs[t % 2],
                out_hbm.at[pl.ds((bi * CT + row0 + t * R) * K, R * K)],
                out_sems[t % 2])

        start_in(0)
        for t in range(NT):
            if t + 1 < NT:
                start_in(t + 1)
            in_cp[t].wait()
            if t >= 2:
                out_cp[t - 2].wait()
            inb, oub = inbufs[t % 2], oubufs[t % 2]

            def row_body(r4, _):
                r = r4 * 4
                rv = jnp.zeros((16,), jnp.int32) + r * V
                for m in range(4):
                    rvm = rv + m * V
                    for j in range(NJ):
                        g = plsc.load_gather(inb, [rvm + cols[j]])
                        oub[pl.ds((r + m) * K + 16 * j, 16)] = g * valv[j]
                return 0

            lax.fori_loop(0, R // 4, row_body, 0)
            start_out(t)
        out_cp[NT - 2].wait()
        out_cp[NT - 1].wait()

    return sc_feature


def _feature_body(idx_ref, vals_ref, x_ref, out_ref, *, K):
    x = x_ref[0]
    V = x.shape[-1]
    idx = idx_ref[0, 0]
    vals = vals_ref[0, 0]
    onehot = (lax.broadcasted_iota(jnp.int32, (V, K), 0) == idx[None, :])
    S = jnp.where(onehot, vals[None, :], 0.0).astype(jnp.bfloat16)
    out_ref[0] = jax.lax.dot_general(
        x, S, (((1,), (0,)), ((), ())),
        preferred_element_type=jnp.float32)


def kernel(X, A, W, b):
    B, C, T, V = X.shape
    K = max(2, V // 2)
    CT = C * T
    RB = 16384
    NCT = CT // RB

    X2 = X.reshape(B, CT, V)
    CB = 256
    NC = C // CB

    X_avg, Xbf = pl.pallas_call(
        functools.partial(_mean_body, T=T),
        grid=(B, NC),
        in_specs=[
            pl.BlockSpec((1, CB, T, V), lambda i, j: (i, j, 0, 0)),
        ],
        out_specs=[
            pl.BlockSpec((1, CB, V), lambda i, j: (i, j, 0)),
            pl.BlockSpec((1, CB, T, V), lambda i, j: (i, j, 0, 0)),
        ],
        out_shape=[
            jax.ShapeDtypeStruct((B, C, V), jnp.float32),
            jax.ShapeDtypeStruct((B, C, T, V), jnp.bfloat16),
        ],
    )(X)
    Z = jnp.transpose(X_avg, (0, 2, 1))
    scores = jax.nn.sigmoid(jnp.squeeze(Z @ W.T + b))

    vals, idx, a_pooled = pl.pallas_call(
        functools.partial(_select_body, K=K),
        out_shape=[
            jax.ShapeDtypeStruct((B, K), jnp.float32),
            jax.ShapeDtypeStruct((B, K), jnp.int32),
            jax.ShapeDtypeStruct((B, K, K), jnp.float32),
        ],
    )(scores, A)

    idx3 = idx.reshape(B, 1, K)
    vals3 = vals.reshape(B, 1, K)
    feats = pl.pallas_call(
        functools.partial(_feature_body, K=K),
        grid=(B, NCT),
        in_specs=[
            pl.BlockSpec((1, 1, K), lambda i, j: (i, 0, 0)),
            pl.BlockSpec((1, 1, K), lambda i, j: (i, 0, 0)),
            pl.BlockSpec((1, RB, V), lambda i, j: (i, j, 0)),
        ],
        out_specs=pl.BlockSpec((1, RB, K), lambda i, j: (i, j, 0)),
        out_shape=jax.ShapeDtypeStruct((B, CT, K), jnp.float32),
    )(idx3, vals3, Xbf.reshape(B, CT, V))

    scaled_features = feats.reshape(B, C, T, K)
    return (a_pooled, scaled_features, idx)


# speedup vs baseline: 1.0128x; 1.0128x over previous
import functools

import jax
import jax.numpy as jnp
from jax import lax
from jax.experimental import pallas as pl
from jax.experimental.pallas import tpu as pltpu


def _mean_body(x_ref, out_ref, xbf_ref, *, T):
    x = x_ref[0]
    out_ref[0] = jnp.sum(x, axis=1) * (1.0 / T)
    xbf_ref[0] = x.astype(jnp.bfloat16)


def _select_body(s_ref, a_ref, vals_ref, idx_ref, apool_ref, *, K):
    s = s_ref[...]
    B, V = s.shape

    su = s[:, :, None]
    sv = s[:, None, :]
    u_iota = lax.broadcasted_iota(jnp.int32, (B, V, V), 1)
    v_iota = lax.broadcasted_iota(jnp.int32, (B, V, V), 2)
    beats = (su > sv) | ((su == sv) & (u_iota < v_iota))
    rank = jnp.sum(beats.astype(jnp.float32), axis=1)
    keep = rank < K

    rank_i = rank.astype(jnp.int32)
    r_iota = lax.broadcasted_iota(jnp.int32, (B, V, K), 2)
    hit = rank_i[:, :, None] == r_iota
    vals_ref[...] = jnp.sum(jnp.where(hit, s[:, :, None], 0.0), axis=1)

    tri = (lax.broadcasted_iota(jnp.int32, (V, V), 0)
           <= lax.broadcasted_iota(jnp.int32, (V, V), 1)).astype(jnp.float32)
    keep_f = keep.astype(jnp.float32)
    slot = jax.lax.dot_general(
        keep_f, tri, (((1,), (0,)), ((), ())),
        preferred_element_type=jnp.float32).astype(jnp.int32) - 1
    k_iota = lax.broadcasted_iota(jnp.int32, (B, V, K), 2)
    sel = (rank_i[:, :, None] < K) & (slot[:, :, None] == k_iota)
    v_in_iota = lax.broadcasted_iota(jnp.int32, (B, V, K), 1)
    idx_ref[...] = jnp.sum(jnp.where(sel, v_in_iota, 0), axis=1)

    selT = (rank_i[:, None, :] < K) & (slot[:, None, :] == lax.broadcasted_iota(
        jnp.int32, (B, K, V), 1))
    sel_f = sel.astype(jnp.float32)
    selT_f = selT.astype(jnp.float32)
    for bi in range(B):
        a_b = a_ref[bi]
        rows = jax.lax.dot_general(
            selT_f[bi], a_b, (((1,), (0,)), ((), ())),
            preferred_element_type=jnp.float32,
            precision=jax.lax.Precision.HIGHEST)
        apool_ref[bi] = jax.lax.dot_general(
            rows, sel_f[bi], (((1,), (0,)), ((), ())),
            preferred_element_type=jnp.float32,
            precision=jax.lax.Precision.HIGHEST)


def _feature_body(idx_ref, vals_ref, x_ref, out_ref, *, K):
    x = x_ref[0]
    V = x.shape[-1]
    idx = idx_ref[0, 0]
    vals = vals_ref[0, 0]
    onehot = (lax.broadcasted_iota(jnp.int32, (V, K), 0) == idx[None, :])
    S = jnp.where(onehot, vals[None, :], 0.0).astype(jnp.bfloat16)
    out_ref[0] = jax.lax.dot_general(
        x, S, (((1,), (0,)), ((), ())),
        preferred_element_type=jnp.float32)


def kernel(X, A, W, b):
    B, C, T, V = X.shape
    K = max(2, V // 2)
    CT = C * T
    RB = 16384
    NCT = CT // RB

    X2 = X.reshape(B, CT, V)
    CB = 128
    NC = C // CB

    X_avg, Xbf = pl.pallas_call(
        functools.partial(_mean_body, T=T),
        grid=(B, NC),
        in_specs=[
            pl.BlockSpec((1, CB, T, V), lambda i, j: (i, j, 0, 0)),
        ],
        out_specs=[
            pl.BlockSpec((1, CB, V), lambda i, j: (i, j, 0)),
            pl.BlockSpec((1, CB, T, V), lambda i, j: (i, j, 0, 0)),
        ],
        out_shape=[
            jax.ShapeDtypeStruct((B, C, V), jnp.float32),
            jax.ShapeDtypeStruct((B, C, T, V), jnp.bfloat16),
        ],
    )(X)
    Z = jnp.transpose(X_avg, (0, 2, 1))
    scores = jax.nn.sigmoid(jnp.squeeze(Z @ W.T + b))

    vals, idx, a_pooled = pl.pallas_call(
        functools.partial(_select_body, K=K),
        out_shape=[
            jax.ShapeDtypeStruct((B, K), jnp.float32),
            jax.ShapeDtypeStruct((B, K), jnp.int32),
            jax.ShapeDtypeStruct((B, K, K), jnp.float32),
        ],
    )(scores, A)

    idx3 = idx.reshape(B, 1, K)
    vals3 = vals.reshape(B, 1, K)
    feats = pl.pallas_call(
        functools.partial(_feature_body, K=K),
        grid=(B, NCT),
        in_specs=[
            pl.BlockSpec((1, 1, K), lambda i, j: (i, 0, 0)),
            pl.BlockSpec((1, 1, K), lambda i, j: (i, 0, 0)),
            pl.BlockSpec((1, RB, V), lambda i, j: (i, j, 0)),
        ],
        out_specs=pl.BlockSpec((1, RB, K), lambda i, j: (i, j, 0)),
        out_shape=jax.ShapeDtypeStruct((B, CT, K), jnp.float32),
    )(idx3, vals3, Xbf.reshape(B, CT, V))

    scaled_features = feats.reshape(B, C, T, K)
    return (a_pooled, scaled_features, idx)


# speedup vs baseline: 1.0141x; 1.0013x over previous
import functools

import jax
import jax.numpy as jnp
from jax import lax
from jax.experimental import pallas as pl


def _mean_body(x_ref, out_ref, xbf_ref, *, T):
    x = x_ref[0]
    out_ref[0] = jnp.sum(x, axis=1) * (1.0 / T)
    xbf_ref[0] = x.astype(jnp.bfloat16)


def _select_body(s_ref, a_ref, vals_ref, idx_ref, apool_ref, *, K):
    s = s_ref[...]
    B, V = s.shape

    su = s[:, :, None]
    sv = s[:, None, :]
    u_iota = lax.broadcasted_iota(jnp.int32, (B, V, V), 1)
    v_iota = lax.broadcasted_iota(jnp.int32, (B, V, V), 2)
    beats = (su > sv) | ((su == sv) & (u_iota < v_iota))
    rank = jnp.sum(beats.astype(jnp.float32), axis=1)
    keep = rank < K

    rank_i = rank.astype(jnp.int32)
    r_iota = lax.broadcasted_iota(jnp.int32, (B, V, K), 2)
    hit = rank_i[:, :, None] == r_iota
    vals_ref[...] = jnp.sum(jnp.where(hit, s[:, :, None], 0.0), axis=1)

    tri = (lax.broadcasted_iota(jnp.int32, (V, V), 0)
           <= lax.broadcasted_iota(jnp.int32, (V, V), 1)).astype(jnp.float32)
    keep_f = keep.astype(jnp.float32)
    slot = jax.lax.dot_general(
        keep_f, tri, (((1,), (0,)), ((), ())),
        preferred_element_type=jnp.float32).astype(jnp.int32) - 1
    k_iota = lax.broadcasted_iota(jnp.int32, (B, V, K), 2)
    sel = (rank_i[:, :, None] < K) & (slot[:, :, None] == k_iota)
    v_in_iota = lax.broadcasted_iota(jnp.int32, (B, V, K), 1)
    idx_ref[...] = jnp.sum(jnp.where(sel, v_in_iota, 0), axis=1)

    selT = (rank_i[:, None, :] < K) & (slot[:, None, :] == lax.broadcasted_iota(
        jnp.int32, (B, K, V), 1))
    sel_f = sel.astype(jnp.float32)
    selT_f = selT.astype(jnp.float32)
    for bi in range(B):
        a_b = a_ref[bi]
        rows = jax.lax.dot_general(
            selT_f[bi], a_b, (((1,), (0,)), ((), ())),
            preferred_element_type=jnp.float32,
            precision=jax.lax.Precision.HIGHEST)
        apool_ref[bi] = jax.lax.dot_general(
            rows, sel_f[bi], (((1,), (0,)), ((), ())),
            preferred_element_type=jnp.float32,
            precision=jax.lax.Precision.HIGHEST)


def _feature_body(idx_ref, vals_ref, x_ref, out_ref, *, K):
    x = x_ref[0]
    V = x.shape[-1]
    idx = idx_ref[0, 0]
    vals = vals_ref[0, 0]
    onehot = (lax.broadcasted_iota(jnp.int32, (V, K), 0) == idx[None, :])
    S = jnp.where(onehot, vals[None, :], 0.0).astype(jnp.bfloat16)
    out_ref[0] = jax.lax.dot_general(
        x, S, (((1,), (0,)), ((), ())),
        preferred_element_type=jnp.float32)


def kernel(X, A, W, b):
    B, C, T, V = X.shape
    K = max(2, V // 2)
    CT = C * T
    RB = 16384
    NCT = CT // RB

    X2 = X.reshape(B, CT, V)
    CB = 128
    NC = C // CB

    X_avg, Xbf = pl.pallas_call(
        functools.partial(_mean_body, T=T),
        grid=(B, NC),
        in_specs=[
            pl.BlockSpec((1, CB, T, V), lambda i, j: (i, j, 0, 0)),
        ],
        out_specs=[
            pl.BlockSpec((1, CB, V), lambda i, j: (i, j, 0)),
            pl.BlockSpec((1, CB, T, V), lambda i, j: (i, j, 0, 0)),
        ],
        out_shape=[
            jax.ShapeDtypeStruct((B, C, V), jnp.float32),
            jax.ShapeDtypeStruct((B, C, T, V), jnp.bfloat16),
        ],
    )(X)
    Z = jnp.transpose(X_avg, (0, 2, 1))
    scores = jax.nn.sigmoid(jnp.squeeze(Z @ W.T + b))

    vals, idx, a_pooled = pl.pallas_call(
        functools.partial(_select_body, K=K),
        out_shape=[
            jax.ShapeDtypeStruct((B, K), jnp.float32),
            jax.ShapeDtypeStruct((B, K), jnp.int32),
            jax.ShapeDtypeStruct((B, K, K), jnp.float32),
        ],
    )(scores, A)

    idx3 = idx.reshape(B, 1, K)
    vals3 = vals.reshape(B, 1, K)
    feats = pl.pallas_call(
        functools.partial(_feature_body, K=K),
        grid=(B, NCT),
        in_specs=[
            pl.BlockSpec((1, 1, K), lambda i, j: (i, 0, 0)),
            pl.BlockSpec((1, 1, K), lambda i, j: (i, 0, 0)),
            pl.BlockSpec((1, RB, V), lambda i, j: (i, j, 0)),
        ],
        out_specs=pl.BlockSpec((1, RB, K), lambda i, j: (i, j, 0)),
        out_shape=jax.ShapeDtypeStruct((B, CT, K), jnp.float32),
    )(idx3, vals3, Xbf.reshape(B, CT, V))

    scaled_features = feats.reshape(B, C, T, K)
    return (a_pooled, scaled_features, idx)
